# baseline (speedup 1.0000x reference)
import functools
import math

import jax
import jax.numpy as jnp
from jax.experimental import pallas as pl
from jax.experimental.pallas import tpu as pltpu

_LANES = 128
_CH = 16


def _round_up(x, m):
    return ((x + m - 1) // m) * m


def _stem_kernel(x_ref, s_ref, wt_ref, bt_ref, o_ref, q_ref, rhs_a, rhs_b, acc_ref,
                 *, C, H, W, Ho, Wo, c_out):
    xb = x_ref[0].reshape(C * H, W)
    q = jnp.dot(xb, s_ref[...], preferred_element_type=jnp.float32)
    q_ref[...] = q.astype(jnp.bfloat16)

    acc_ref[...] = jnp.zeros_like(acc_ref)
    rhs_a[...] = jnp.zeros_like(rhs_a)
    rhs_b[...] = jnp.zeros_like(rhs_b)

    CHB = 2 * _CH

    def assemble(ch_base, rhs_ref, first):
        off = 0 if first else 16
        for c in range(C):
            if first:
                qc = q_ref[c * H:c * H + CHB + 16, :]
            else:
                qc = q_ref[pl.ds(ch_base + c * H, CHB + 16), :]
            for t in range(-1, CHB):
                if first and t < 0:
                    continue
                row = qc[off + t:off + t + 1, :]
                for di in range(3):
                    num = t + 1 - di
                    if num % 2 or not 0 <= num // 2 < _CH:
                        continue
                    dst = (num // 2) * _LANES
                    for dj in range(3):
                        k = di * 9 + dj * 3 + c
                        rhs_ref[k:k + 1, dst:dst + _LANES] = \
                            row[:, dj * _LANES:(dj + 1) * _LANES]

    def dotacc(rhs_ref):
        y = jnp.dot(wt_ref[...], rhs_ref[...], preferred_element_type=jnp.float32)
        y = y[0:c_out, :] + bt_ref[...]
        acc_ref[...] += y * jax.nn.sigmoid(y)

    n_chunks = Ho // _CH
    assemble(0, rhs_a, True)
    if n_chunks == 1:
        dotacc(rhs_a)
    else:
        n_iter = (n_chunks - 2) // 2

        def body2(i, _):
            b1 = pl.multiple_of(2 * CHB * i + CHB - 16, 16)
            b2 = pl.multiple_of(2 * CHB * i + 2 * CHB - 16, 16)
            assemble(b1, rhs_b, False)
            dotacc(rhs_a)
            assemble(b2, rhs_a, False)
            dotacc(rhs_b)
            return 0

        jax.lax.fori_loop(0, n_iter, body2, 0)
        if n_chunks - 1 - 2 * n_iter == 2:
            assemble(CHB * (n_chunks - 2) - 16, rhs_b, False)
            dotacc(rhs_a)
            assemble(CHB * (n_chunks - 1) - 16, rhs_a, False)
            dotacc(rhs_b)
            dotacc(rhs_a)
        else:
            assemble(CHB * (n_chunks - 1) - 16, rhs_b, False)
            dotacc(rhs_a)
            dotacc(rhs_b)

    lane = jax.lax.broadcasted_iota(jnp.int32, (c_out, _CH * _LANES), 1) % _LANES
    pooled = jnp.sum(jnp.where(lane < Wo, acc_ref[...], 0.0),
                     axis=1, keepdims=True) * (1.0 / (Ho * Wo))
    o_ref[0] = pooled


def _erf_poly(x):
    a1, a2, a3, a4, a5 = 0.254829592, -0.284496736, 1.421413741, -1.453152027, 1.061405429
    p = 0.3275911
    s = jnp.where(x >= 0.0, 1.0, -1.0)
    z = jnp.abs(x)
    t = 1.0 / (1.0 + p * z)
    poly = t * (a1 + t * (a2 + t * (a3 + t * (a4 + t * a5))))
    return s * (1.0 - poly * jnp.exp(-z * z))


def _gelu(x):
    return 0.5 * x * (1.0 + _erf_poly(x * 0.7071067811865476))


def _head_kernel(x_ref, wa_ref, ba_ref, wb_ref, bb_ref, wc_ref, bc_ref, o_ref):
    h = jnp.dot(x_ref[...], wa_ref[...], preferred_element_type=jnp.float32) + ba_ref[...]
    h = _gelu(h)
    h = jnp.dot(h.astype(jnp.bfloat16), wb_ref[...],
                preferred_element_type=jnp.float32) + bb_ref[...]
    h = _gelu(h)
    o_ref[...] = jnp.dot(h.astype(jnp.bfloat16), wc_ref[...],
                         preferred_element_type=jnp.float32) + bc_ref[...]


def _col_select(W, Wo):
    j = jnp.arange(W)[:, None]
    col = jnp.arange(3 * _LANES)[None, :]
    ow = col % _LANES
    dj = col // _LANES
    sel = (ow < Wo) & (j == 2 * ow + dj - 1)
    return sel.astype(jnp.bfloat16)


def _stem_pool(x, w_stem, b_stem):
    B, C, H, W = x.shape
    Ho, Wo = (H + 1) // 2, (W + 1) // 2
    c_out = 48
    kdim = _round_up(3 * 3 * C, 32)

    Wp = _round_up(W, _LANES)
    x = jnp.pad(x, ((0, 0), (0, 0), (0, 0), (0, Wp - W))).astype(jnp.bfloat16)
    W = Wp

    s = _col_select(W, Wo)
    wt = jnp.pad(w_stem.T, ((0, 0), (0, kdim - w_stem.shape[0])))
    bt = b_stem[0, 0:c_out].reshape(c_out, 1)

    kern = functools.partial(_stem_kernel, C=C, H=H, W=W, Ho=Ho, Wo=Wo, c_out=c_out)
    out = pl.pallas_call(
        kern,
        out_shape=jax.ShapeDtypeStruct((B, c_out, 1), jnp.float32),
        grid=(B,),
        in_specs=[
            pl.BlockSpec((1, C, H, W), lambda b: (b, 0, 0, 0)),
            pl.BlockSpec((W, 3 * _LANES), lambda b: (0, 0)),
            pl.BlockSpec((_LANES, kdim), lambda b: (0, 0)),
            pl.BlockSpec((c_out, 1), lambda b: (0, 0)),
        ],
        out_specs=pl.BlockSpec((1, c_out, 1), lambda b: (b, 0, 0)),
        scratch_shapes=[
            pltpu.VMEM((C * H, 3 * _LANES), jnp.bfloat16),
            pltpu.VMEM((kdim, _CH * _LANES), jnp.bfloat16),
            pltpu.VMEM((kdim, _CH * _LANES), jnp.bfloat16),
            pltpu.VMEM((c_out, _CH * _LANES), jnp.float32),
        ],
        compiler_params=pltpu.CompilerParams(
            dimension_semantics=("parallel",),
            vmem_limit_bytes=32 * 1024 * 1024),
    )(x, s, wt, bt)
    return out[:, :, 0]


def _head(pooled48, wa, ba, wb, bb, wc, bc):
    B = pooled48.shape[0]
    x48 = pooled48.astype(jnp.bfloat16)
    wa48 = wa[0:48, :]
    args = (x48, wa48, ba, wb, bb, wc, bc)
    spec = pl.BlockSpec(memory_space=pltpu.MemorySpace.VMEM)
    out = pl.pallas_call(
        _head_kernel,
        out_shape=jax.ShapeDtypeStruct((B, _LANES), jnp.float32),
        in_specs=[spec] * len(args),
        out_specs=spec,
        compiler_params=pltpu.CompilerParams(vmem_limit_bytes=32 * 1024 * 1024),
    )(*args)
    return out


@jax.jit
def _forward(x, w_stem, b_stem, wa, ba, wb, bb, wc, bc):
    pooled = _stem_pool(x, w_stem, b_stem)
    return _head(pooled, wa, ba, wb, bb, wc, bc)[:, :8]


def kernel(x, w_stem, b_stem, wa, ba, wb, bb, wc, bc):
    return _forward(x, w_stem, b_stem, wa, ba, wb, bb, wc, bc)


# speedup vs baseline: 1.1567x; 1.1567x over previous
import functools
import math

import jax
import jax.numpy as jnp
from jax.experimental import pallas as pl
from jax.experimental.pallas import tpu as pltpu

_LANES = 128
_CH = 16


def _round_up(x, m):
    return ((x + m - 1) // m) * m


def _stem_kernel(x_ref, s_ref, wt_ref, bt_ref, o_ref, q_ref, rhs_a, rhs_b, acc_ref,
                 *, C, H, W, Ho, Wo, c_out, n_img):
    rhs_a[...] = jnp.zeros_like(rhs_a)
    rhs_b[...] = jnp.zeros_like(rhs_b)

    CHB = 2 * _CH

    def assemble(ch_base, rhs_ref, first):
        off = 0 if first else 16
        zero_row = jnp.zeros((1, _LANES), jnp.bfloat16)
        for c in range(C):
            if first:
                qc = q_ref[c * H:c * H + CHB + 16, :]
                for dj in range(3):
                    rhs_ref[dj * 3 + c:dj * 3 + c + 1, 0:_LANES] = zero_row
            else:
                qc = q_ref[pl.ds(ch_base + c * H, CHB + 16), :]
            for t in range(-1, CHB):
                if first and t < 0:
                    continue
                row = qc[off + t:off + t + 1, :]
                for di in range(3):
                    num = t + 1 - di
                    if num % 2 or not 0 <= num // 2 < _CH:
                        continue
                    dst = (num // 2) * _LANES
                    for dj in range(3):
                        k = di * 9 + dj * 3 + c
                        rhs_ref[k:k + 1, dst:dst + _LANES] = \
                            row[:, dj * _LANES:(dj + 1) * _LANES]

    def dotacc(rhs_ref):
        y = jnp.dot(wt_ref[...], rhs_ref[...], preferred_element_type=jnp.float32)
        y = y[0:c_out, :] + bt_ref[...]
        acc_ref[...] += y * jax.nn.sigmoid(y)

    n_chunks = Ho // _CH
    for img in range(n_img):
        xb = x_ref[img].astype(jnp.bfloat16).reshape(C * H, W)
        q = jnp.dot(xb, s_ref[...], preferred_element_type=jnp.float32)
        q_ref[...] = q.astype(jnp.bfloat16)
        acc_ref[...] = jnp.zeros_like(acc_ref)

        assemble(0, rhs_a, True)
        if n_chunks == 1:
            dotacc(rhs_a)
        else:
            n_iter = (n_chunks - 2) // 2

            def body2(i, _):
                b1 = pl.multiple_of(2 * CHB * i + CHB - 16, 16)
                b2 = pl.multiple_of(2 * CHB * i + 2 * CHB - 16, 16)
                assemble(b1, rhs_b, False)
                dotacc(rhs_a)
                assemble(b2, rhs_a, False)
                dotacc(rhs_b)
                return 0

            jax.lax.fori_loop(0, n_iter, body2, 0)
            if n_chunks - 1 - 2 * n_iter == 2:
                assemble(CHB * (n_chunks - 2) - 16, rhs_b, False)
                dotacc(rhs_a)
                assemble(CHB * (n_chunks - 1) - 16, rhs_a, False)
                dotacc(rhs_b)
                dotacc(rhs_a)
            else:
                assemble(CHB * (n_chunks - 1) - 16, rhs_b, False)
                dotacc(rhs_a)
                dotacc(rhs_b)

        lane = jax.lax.broadcasted_iota(jnp.int32, (c_out, _CH * _LANES), 1) % _LANES
        pooled = jnp.sum(jnp.where(lane < Wo, acc_ref[...], 0.0),
                         axis=1, keepdims=True) * (1.0 / (Ho * Wo))
        o_ref[img] = pooled


def _erf_poly(x):
    a1, a2, a3, a4, a5 = 0.254829592, -0.284496736, 1.421413741, -1.453152027, 1.061405429
    p = 0.3275911
    s = jnp.where(x >= 0.0, 1.0, -1.0)
    z = jnp.abs(x)
    t = 1.0 / (1.0 + p * z)
    poly = t * (a1 + t * (a2 + t * (a3 + t * (a4 + t * a5))))
    return s * (1.0 - poly * jnp.exp(-z * z))


def _gelu(x):
    return 0.5 * x * (1.0 + _erf_poly(x * 0.7071067811865476))


def _head_kernel(x_ref, wa_ref, ba_ref, wb_ref, bb_ref, wc_ref, bc_ref, o_ref):
    h = jnp.dot(x_ref[...], wa_ref[...], preferred_element_type=jnp.float32) + ba_ref[...]
    h = _gelu(h)
    h = jnp.dot(h.astype(jnp.bfloat16), wb_ref[...],
                preferred_element_type=jnp.float32) + bb_ref[...]
    h = _gelu(h)
    o_ref[...] = jnp.dot(h.astype(jnp.bfloat16), wc_ref[...],
                         preferred_element_type=jnp.float32) + bc_ref[...]


def _col_select(W, Wo):
    j = jnp.arange(W)[:, None]
    col = jnp.arange(3 * _LANES)[None, :]
    ow = col % _LANES
    dj = col // _LANES
    sel = (ow < Wo) & (j == 2 * ow + dj - 1)
    return sel.astype(jnp.bfloat16)


def _stem_pool(x, w_stem, b_stem):
    B, C, H, W = x.shape
    Ho, Wo = (H + 1) // 2, (W + 1) // 2
    c_out = 48
    kdim = _round_up(3 * 3 * C, 32)

    s = _col_select(W, Wo)
    wt = jnp.pad(w_stem.T, ((0, 0), (0, kdim - w_stem.shape[0])))
    bt = b_stem[0, 0:c_out].reshape(c_out, 1)

    n_img = 2 if B % 2 == 0 else 1
    kern = functools.partial(_stem_kernel, C=C, H=H, W=W, Ho=Ho, Wo=Wo,
                             c_out=c_out, n_img=n_img)
    out = pl.pallas_call(
        kern,
        out_shape=jax.ShapeDtypeStruct((B, c_out, 1), jnp.float32),
        grid=(B // n_img,),
        in_specs=[
            pl.BlockSpec((n_img, C, H, W), lambda b: (b, 0, 0, 0)),
            pl.BlockSpec((W, 3 * _LANES), lambda b: (0, 0)),
            pl.BlockSpec((_LANES, kdim), lambda b: (0, 0)),
            pl.BlockSpec((c_out, 1), lambda b: (0, 0)),
        ],
        out_specs=pl.BlockSpec((n_img, c_out, 1), lambda b: (b, 0, 0)),
        scratch_shapes=[
            pltpu.VMEM((C * H, 3 * _LANES), jnp.bfloat16),
            pltpu.VMEM((kdim, _CH * _LANES), jnp.bfloat16),
            pltpu.VMEM((kdim, _CH * _LANES), jnp.bfloat16),
            pltpu.VMEM((c_out, _CH * _LANES), jnp.float32),
        ],
        compiler_params=pltpu.CompilerParams(
            dimension_semantics=("parallel",),
            vmem_limit_bytes=32 * 1024 * 1024),
    )(x, s, wt, bt)
    return out[:, :, 0]


def _head(pooled48, wa, ba, wb, bb, wc, bc):
    B = pooled48.shape[0]
    x48 = pooled48.astype(jnp.bfloat16)
    wa48 = wa[0:48, :]
    args = (x48, wa48, ba, wb, bb, wc, bc)
    spec = pl.BlockSpec(memory_space=pltpu.MemorySpace.VMEM)
    out = pl.pallas_call(
        _head_kernel,
        out_shape=jax.ShapeDtypeStruct((B, _LANES), jnp.float32),
        in_specs=[spec] * len(args),
        out_specs=spec,
        compiler_params=pltpu.CompilerParams(vmem_limit_bytes=32 * 1024 * 1024),
    )(*args)
    return out


@jax.jit
def _forward(x, w_stem, b_stem, wa, ba, wb, bb, wc, bc):
    pooled = _stem_pool(x, w_stem, b_stem)
    return _head(pooled, wa, ba, wb, bb, wc, bc)[:, :8]


def kernel(x, w_stem, b_stem, wa, ba, wb, bb, wc, bc):
    return _forward(x, w_stem, b_stem, wa, ba, wb, bb, wc, bc)


# speedup vs baseline: 1.1686x; 1.0103x over previous
import functools
import math

import jax
import jax.numpy as jnp
from jax.experimental import pallas as pl
from jax.experimental.pallas import tpu as pltpu

_LANES = 128
_CH = 16


def _round_up(x, m):
    return ((x + m - 1) // m) * m


def _stem_kernel(x_ref, s_ref, wt_ref, bt_ref, o_ref, q_ref, rhs_a, rhs_b, acc_ref,
                 *, C, H, W, Ho, Wo, c_out, n_img):
    rhs_a[...] = jnp.zeros_like(rhs_a)
    rhs_b[...] = jnp.zeros_like(rhs_b)

    CHB = 2 * _CH

    def assemble(ch_base, rhs_ref, first):
        off = 0 if first else 16
        zero_row = jnp.zeros((1, _LANES), jnp.bfloat16)
        for c in range(C):
            if first:
                qc = q_ref[c * H:c * H + CHB + 16, :]
                for dj in range(3):
                    rhs_ref[dj * 3 + c:dj * 3 + c + 1, 0:_LANES] = zero_row
            else:
                qc = q_ref[pl.ds(ch_base + c * H, CHB + 16), :]
            for t in range(-1, CHB):
                if first and t < 0:
                    continue
                row = qc[off + t:off + t + 1, :]
                for di in range(3):
                    num = t + 1 - di
                    if num % 2 or not 0 <= num // 2 < _CH:
                        continue
                    dst = (num // 2) * _LANES
                    for dj in range(3):
                        k = di * 9 + dj * 3 + c
                        rhs_ref[k:k + 1, dst:dst + _LANES] = \
                            row[:, dj * _LANES:(dj + 1) * _LANES]

    def dotacc(rhs_ref):
        y = jnp.dot(wt_ref[...], rhs_ref[...], preferred_element_type=jnp.float32)
        y = y[0:c_out, :] + bt_ref[...]
        acc_ref[...] += y * jax.nn.sigmoid(y)

    n_chunks = Ho // _CH
    for img in range(n_img):
        xb = x_ref[img].astype(jnp.bfloat16).reshape(C * H, W)
        q = jnp.dot(xb, s_ref[...], preferred_element_type=jnp.float32)
        q_ref[...] = q.astype(jnp.bfloat16)
        acc_ref[...] = jnp.zeros_like(acc_ref)

        assemble(0, rhs_a, True)
        if n_chunks == 1:
            dotacc(rhs_a)
        else:
            n_iter = (n_chunks - 2) // 2

            def body2(i, _):
                b1 = pl.multiple_of(2 * CHB * i + CHB - 16, 16)
                b2 = pl.multiple_of(2 * CHB * i + 2 * CHB - 16, 16)
                assemble(b1, rhs_b, False)
                dotacc(rhs_a)
                assemble(b2, rhs_a, False)
                dotacc(rhs_b)
                return 0

            jax.lax.fori_loop(0, n_iter, body2, 0)
            if n_chunks - 1 - 2 * n_iter == 2:
                assemble(CHB * (n_chunks - 2) - 16, rhs_b, False)
                dotacc(rhs_a)
                assemble(CHB * (n_chunks - 1) - 16, rhs_a, False)
                dotacc(rhs_b)
                dotacc(rhs_a)
            else:
                assemble(CHB * (n_chunks - 1) - 16, rhs_b, False)
                dotacc(rhs_a)
                dotacc(rhs_b)

        lane = jax.lax.broadcasted_iota(jnp.int32, (c_out, _CH * _LANES), 1) % _LANES
        pooled = jnp.sum(jnp.where(lane < Wo, acc_ref[...], 0.0),
                         axis=1, keepdims=True) * (1.0 / (Ho * Wo))
        o_ref[img] = pooled


def _erf_poly(x):
    a1, a2, a3, a4, a5 = 0.254829592, -0.284496736, 1.421413741, -1.453152027, 1.061405429
    p = 0.3275911
    s = jnp.where(x >= 0.0, 1.0, -1.0)
    z = jnp.abs(x)
    t = 1.0 / (1.0 + p * z)
    poly = t * (a1 + t * (a2 + t * (a3 + t * (a4 + t * a5))))
    return s * (1.0 - poly * jnp.exp(-z * z))


def _gelu(x):
    return 0.5 * x * (1.0 + _erf_poly(x * 0.7071067811865476))


def _head_kernel(x_ref, wa_ref, ba_ref, wb_ref, bb_ref, wc_ref, bc_ref, o_ref):
    h = jnp.dot(x_ref[...], wa_ref[...], preferred_element_type=jnp.float32) + ba_ref[...]
    h = _gelu(h)
    h = jnp.dot(h.astype(jnp.bfloat16), wb_ref[...],
                preferred_element_type=jnp.float32) + bb_ref[...]
    h = _gelu(h)
    o_ref[...] = jnp.dot(h.astype(jnp.bfloat16), wc_ref[...],
                         preferred_element_type=jnp.float32) + bc_ref[...]


def _col_select(W, Wo):
    j = jnp.arange(W)[:, None]
    col = jnp.arange(3 * _LANES)[None, :]
    ow = col % _LANES
    dj = col // _LANES
    sel = (ow < Wo) & (j == 2 * ow + dj - 1)
    return sel.astype(jnp.bfloat16)


def _stem_pool(x, w_stem, b_stem):
    B, C, H, W = x.shape
    Ho, Wo = (H + 1) // 2, (W + 1) // 2
    c_out = 48
    kdim = _round_up(3 * 3 * C, 32)

    s = _col_select(W, Wo)
    wt = jnp.pad(w_stem.T, ((0, 0), (0, kdim - w_stem.shape[0])))
    bt = b_stem[0, 0:c_out].reshape(c_out, 1)

    n_img = 4 if B % 4 == 0 else (2 if B % 2 == 0 else 1)
    kern = functools.partial(_stem_kernel, C=C, H=H, W=W, Ho=Ho, Wo=Wo,
                             c_out=c_out, n_img=n_img)
    out = pl.pallas_call(
        kern,
        out_shape=jax.ShapeDtypeStruct((B, c_out, 1), jnp.float32),
        grid=(B // n_img,),
        in_specs=[
            pl.BlockSpec((n_img, C, H, W), lambda b: (b, 0, 0, 0)),
            pl.BlockSpec((W, 3 * _LANES), lambda b: (0, 0)),
            pl.BlockSpec((_LANES, kdim), lambda b: (0, 0)),
            pl.BlockSpec((c_out, 1), lambda b: (0, 0)),
        ],
        out_specs=pl.BlockSpec((n_img, c_out, 1), lambda b: (b, 0, 0)),
        scratch_shapes=[
            pltpu.VMEM((C * H, 3 * _LANES), jnp.bfloat16),
            pltpu.VMEM((kdim, _CH * _LANES), jnp.bfloat16),
            pltpu.VMEM((kdim, _CH * _LANES), jnp.bfloat16),
            pltpu.VMEM((c_out, _CH * _LANES), jnp.float32),
        ],
        compiler_params=pltpu.CompilerParams(
            dimension_semantics=("parallel",),
            vmem_limit_bytes=32 * 1024 * 1024),
    )(x, s, wt, bt)
    return out[:, :, 0]


def _head(pooled48, wa, ba, wb, bb, wc, bc):
    B = pooled48.shape[0]
    x48 = pooled48.astype(jnp.bfloat16)
    wa48 = wa[0:48, :]
    args = (x48, wa48, ba, wb, bb, wc, bc)
    spec = pl.BlockSpec(memory_space=pltpu.MemorySpace.VMEM)
    out = pl.pallas_call(
        _head_kernel,
        out_shape=jax.ShapeDtypeStruct((B, _LANES), jnp.float32),
        in_specs=[spec] * len(args),
        out_specs=spec,
        compiler_params=pltpu.CompilerParams(vmem_limit_bytes=32 * 1024 * 1024),
    )(*args)
    return out


@jax.jit
def _forward(x, w_stem, b_stem, wa, ba, wb, bb, wc, bc):
    pooled = _stem_pool(x, w_stem, b_stem)
    return _head(pooled, wa, ba, wb, bb, wc, bc)[:, :8]


def kernel(x, w_stem, b_stem, wa, ba, wb, bb, wc, bc):
    return _forward(x, w_stem, b_stem, wa, ba, wb, bb, wc, bc)


# speedup vs baseline: 1.1956x; 1.0230x over previous
import functools
import math

import jax
import jax.numpy as jnp
from jax.experimental import pallas as pl
from jax.experimental.pallas import tpu as pltpu

_LANES = 128
_CH = 16


def _round_up(x, m):
    return ((x + m - 1) // m) * m


def _stem_kernel(x_ref, s_ref, wt_ref, bt_ref, o_ref, q_ref, rhs_a, rhs_b, acc_ref,
                 *, C, H, W, Ho, Wo, c_out, n_img):
    rhs_a[...] = jnp.zeros_like(rhs_a)
    rhs_b[...] = jnp.zeros_like(rhs_b)

    CHB = 2 * _CH

    def assemble(ch_base, rhs_ref, first):
        off = 0 if first else 16
        zero_row = jnp.zeros((1, _LANES), jnp.bfloat16)
        for c in range(C):
            if first:
                qc = q_ref[c * H:c * H + CHB + 16, :]
                for dj in range(3):
                    rhs_ref[dj * 3 + c:dj * 3 + c + 1, 0:_LANES] = zero_row
            else:
                qc = q_ref[pl.ds(ch_base + c * H, CHB + 16), :]
            for t in range(-1, CHB):
                if first and t < 0:
                    continue
                row = qc[off + t:off + t + 1, :]
                for di in range(3):
                    num = t + 1 - di
                    if num % 2 or not 0 <= num // 2 < _CH:
                        continue
                    dst = (num // 2) * _LANES
                    for dj in range(3):
                        k = di * 9 + dj * 3 + c
                        rhs_ref[k:k + 1, dst:dst + _LANES] = \
                            row[:, dj * _LANES:(dj + 1) * _LANES]

    def dotacc(rhs_ref):
        y = jnp.dot(wt_ref[...], rhs_ref[...], preferred_element_type=jnp.float32)
        y = y + bt_ref[...]
        acc_ref[...] += y * jax.nn.sigmoid(y)

    n_chunks = Ho // _CH
    for img in range(n_img):
        xb = x_ref[img].astype(jnp.bfloat16).reshape(C * H, W)
        q = jnp.dot(xb, s_ref[...], preferred_element_type=jnp.float32)
        q_ref[...] = q.astype(jnp.bfloat16)
        acc_ref[...] = jnp.zeros_like(acc_ref)

        assemble(0, rhs_a, True)
        if n_chunks == 1:
            dotacc(rhs_a)
        else:
            n_iter = (n_chunks - 2) // 2

            def body2(i, _):
                b1 = pl.multiple_of(2 * CHB * i + CHB - 16, 16)
                b2 = pl.multiple_of(2 * CHB * i + 2 * CHB - 16, 16)
                assemble(b1, rhs_b, False)
                dotacc(rhs_a)
                assemble(b2, rhs_a, False)
                dotacc(rhs_b)
                return 0

            jax.lax.fori_loop(0, n_iter, body2, 0)
            if n_chunks - 1 - 2 * n_iter == 2:
                assemble(CHB * (n_chunks - 2) - 16, rhs_b, False)
                dotacc(rhs_a)
                assemble(CHB * (n_chunks - 1) - 16, rhs_a, False)
                dotacc(rhs_b)
                dotacc(rhs_a)
            else:
                assemble(CHB * (n_chunks - 1) - 16, rhs_b, False)
                dotacc(rhs_a)
                dotacc(rhs_b)

        lane = jax.lax.broadcasted_iota(jnp.int32, (c_out, _CH * _LANES), 1) % _LANES
        pooled = jnp.sum(jnp.where(lane < Wo, acc_ref[...], 0.0),
                         axis=1, keepdims=True) * (1.0 / (Ho * Wo))
        o_ref[img] = pooled


def _erf_poly(x):
    a1, a2, a3, a4, a5 = 0.254829592, -0.284496736, 1.421413741, -1.453152027, 1.061405429
    p = 0.3275911
    s = jnp.where(x >= 0.0, 1.0, -1.0)
    z = jnp.abs(x)
    t = 1.0 / (1.0 + p * z)
    poly = t * (a1 + t * (a2 + t * (a3 + t * (a4 + t * a5))))
    return s * (1.0 - poly * jnp.exp(-z * z))


def _gelu(x):
    return 0.5 * x * (1.0 + _erf_poly(x * 0.7071067811865476))


def _head_kernel(x_ref, wa_ref, ba_ref, wb_ref, bb_ref, wc_ref, bc_ref, o_ref):
    h = jnp.dot(x_ref[...], wa_ref[...], preferred_element_type=jnp.float32) + ba_ref[...]
    h = _gelu(h)
    h = jnp.dot(h.astype(jnp.bfloat16), wb_ref[...],
                preferred_element_type=jnp.float32) + bb_ref[...]
    h = _gelu(h)
    o_ref[...] = jnp.dot(h.astype(jnp.bfloat16), wc_ref[...],
                         preferred_element_type=jnp.float32) + bc_ref[...]


def _col_select(W, Wo):
    j = jnp.arange(W)[:, None]
    col = jnp.arange(3 * _LANES)[None, :]
    ow = col % _LANES
    dj = col // _LANES
    sel = (ow < Wo) & (j == 2 * ow + dj - 1)
    return sel.astype(jnp.bfloat16)


def _stem_pool(x, w_stem, b_stem):
    B, C, H, W = x.shape
    Ho, Wo = (H + 1) // 2, (W + 1) // 2
    c_out = 48
    kdim = _round_up(3 * 3 * C, 32)

    s = _col_select(W, Wo)
    wt = jnp.pad(w_stem.T[0:c_out], ((0, 0), (0, kdim - w_stem.shape[0])))
    bt = b_stem[0, 0:c_out].reshape(c_out, 1)

    n_img = 4 if B % 4 == 0 else (2 if B % 2 == 0 else 1)
    kern = functools.partial(_stem_kernel, C=C, H=H, W=W, Ho=Ho, Wo=Wo,
                             c_out=c_out, n_img=n_img)
    out = pl.pallas_call(
        kern,
        out_shape=jax.ShapeDtypeStruct((B, c_out, 1), jnp.float32),
        grid=(B // n_img,),
        in_specs=[
            pl.BlockSpec((n_img, C, H, W), lambda b: (b, 0, 0, 0)),
            pl.BlockSpec((W, 3 * _LANES), lambda b: (0, 0)),
            pl.BlockSpec((c_out, kdim), lambda b: (0, 0)),
            pl.BlockSpec((c_out, 1), lambda b: (0, 0)),
        ],
        out_specs=pl.BlockSpec((n_img, c_out, 1), lambda b: (b, 0, 0)),
        scratch_shapes=[
            pltpu.VMEM((C * H, 3 * _LANES), jnp.bfloat16),
            pltpu.VMEM((kdim, _CH * _LANES), jnp.bfloat16),
            pltpu.VMEM((kdim, _CH * _LANES), jnp.bfloat16),
            pltpu.VMEM((c_out, _CH * _LANES), jnp.float32),
        ],
        compiler_params=pltpu.CompilerParams(
            dimension_semantics=("parallel",),
            vmem_limit_bytes=32 * 1024 * 1024),
    )(x, s, wt, bt)
    return out[:, :, 0]


def _head(pooled48, wa, ba, wb, bb, wc, bc):
    B = pooled48.shape[0]
    x48 = pooled48.astype(jnp.bfloat16)
    wa48 = wa[0:48, :]
    args = (x48, wa48, ba, wb, bb, wc, bc)
    spec = pl.BlockSpec(memory_space=pltpu.MemorySpace.VMEM)
    out = pl.pallas_call(
        _head_kernel,
        out_shape=jax.ShapeDtypeStruct((B, _LANES), jnp.float32),
        in_specs=[spec] * len(args),
        out_specs=spec,
        compiler_params=pltpu.CompilerParams(vmem_limit_bytes=32 * 1024 * 1024),
    )(*args)
    return out


@jax.jit
def _forward(x, w_stem, b_stem, wa, ba, wb, bb, wc, bc):
    pooled = _stem_pool(x, w_stem, b_stem)
    return _head(pooled, wa, ba, wb, bb, wc, bc)[:, :8]


def kernel(x, w_stem, b_stem, wa, ba, wb, bb, wc, bc):
    return _forward(x, w_stem, b_stem, wa, ba, wb, bb, wc, bc)
